# baseline (speedup 1.0000x reference)
import functools

import jax
import jax.numpy as jnp
from jax import lax
from jax.experimental import pallas as pl
from jax.experimental.pallas import tpu as pltpu
from jax.experimental.pallas import tpu_sc as plsc

B = 4
N = 8192
K = 4096
NLEV = 5
GS = [32, 16, 8, 4, 2]
NS = [8192, 4096, 2048, 1024, 512]
CELL_OFF = [0, 32768, 36864, 37376, 37440]
NCELL = 37448
NCPAD = 37504
SPAN = NCPAD // 8
UPD_ROWS = [n * 8 // 128 for n in NS]
UPD_ROW_OFF = [0, 512, 768, 896, 960]
TOT_ROWS = 992
HALF_ROWS = TOT_ROWS // 2
CHUNK = 16
NCHUNK = HALF_ROWS // CHUNK


def _fps_body(x_ref, sel_ref):
    p = [[x_ref[b, c].reshape(8, 8, 128) for c in range(3)] for b in range(B)]
    i0 = lax.broadcasted_iota(jnp.int32, (8, 8, 128), 0)
    i1 = lax.broadcasted_iota(jnp.int32, (8, 8, 128), 1)
    i2 = lax.broadcasted_iota(jnp.int32, (8, 8, 128), 2)
    flat = i0 * 1024 + i1 * 128 + i2
    big = jnp.int32(1 << 30)
    ones = jnp.ones((128, 128), jnp.float32)

    def lane_sum_bcast(v8):
        s = jnp.dot(v8, ones, preferred_element_type=jnp.float32,
                    precision=jax.lax.Precision.HIGHEST)
        for sft in (4, 2, 1):
            s = s + pltpu.roll(s, sft, 0)
        return s

    carry0 = []
    for b in range(B):
        l0 = [x_ref[b, c, 0, 0] for c in range(3)]
        for c in range(3):
            sel_ref[b, c, pl.ds(0, 1), :] = jnp.full((1, 1), l0[c], jnp.float32)
        carry0.append((jnp.full((8, 8, 128), 1e10, jnp.float32),
                       *[jnp.full((8, 128), l0[c], jnp.float32)
                         for c in range(3)]))

    def body(i, carry):
        new = []
        for b in range(B):
            dists, lx, ly, lz = carry[b]
            dx = p[b][0] - lx[None]
            dd = dx * dx
            dy = p[b][1] - ly[None]
            dd = dd + dy * dy
            dz = p[b][2] - lz[None]
            dd = dd + dz * dz
            dists = jnp.minimum(dists, dd)
            m = jnp.max(dists)
            oh = dists == m
            exs = [jnp.sum(jnp.where(oh, p[b][c], 0.0), axis=0)
                   for c in range(3)]
            cnt8 = jnp.sum(jnp.where(oh, 1.0, 0.0), axis=0)
            sx = lane_sum_bcast(exs[0])
            sy = lane_sum_bcast(exs[1])
            sz = lane_sum_bcast(exs[2])
            scnt = lane_sum_bcast(cnt8)

            def fast(_):
                return sx, sy, sz

            def slow(_):
                iv = jnp.where(oh, flat, big)
                idx = jnp.min(iv)
                oh2 = flat == idx
                return tuple(
                    lane_sum_bcast(jnp.sum(jnp.where(oh2, p[b][c], 0.0),
                                           axis=0))
                    for c in range(3))

            nl = lax.cond(scnt[0, 0] == 1.0, fast, slow, None)
            new.append((dists, *nl))
        for b in range(B):
            for c in range(3):
                sel_ref[b, c, pl.ds(i, 1), :] = new[b][1 + c][0:1, 0:1]
        return tuple(new)

    lax.fori_loop(1, K, body, tuple(carry0))


def _corner_body(x_ref, sel_ref, uidx_ref, vals_ref):
    for l in range(NLEV):
        g = GS[l]
        rows = NS[l] // 128
        if l == 0:
            p = [x_ref[0, c] for c in range(3)]
        else:
            p = [sel_ref[0, c, pl.ds(0, rows), :] for c in range(3)]
        gf = float(g)
        r = [(pc + 0.5) * gf - 0.5 for pc in p]
        lo = [jnp.clip(jnp.floor(rc), 0.0, gf - 1.0) for rc in r]
        hi = [jnp.clip(jnp.ceil(rc), 0.0, gf - 1.0) for rc in r]
        idx_blocks = []
        val_blocks = [[], [], [], []]
        for corner in range(8):
            cx = hi[0] if (corner >> 2) & 1 else lo[0]
            cy = hi[1] if (corner >> 1) & 1 else lo[1]
            cz = hi[2] if corner & 1 else lo[2]
            resx = r[0] - cx
            resy = r[1] - cy
            resz = r[2] - cz
            dist = jnp.sqrt(resx * resx + resy * resy + resz * resz)
            w = (dist < 0.87).astype(jnp.float32)
            cell = (cx * (gf * gf) + cy * gf + cz).astype(jnp.int32) + CELL_OFF[l]
            idx_blocks.append(cell)
            val_blocks[0].append(resx * w)
            val_blocks[1].append(resy * w)
            val_blocks[2].append(resz * w)
            val_blocks[3].append(w)
        ro = UPD_ROW_OFF[l]
        nrow = 8 * rows
        uidx_ref[0, pl.ds(ro, nrow), :] = jnp.concatenate(idx_blocks, axis=0)
        for kind in range(4):
            vals_ref[0, kind, pl.ds(ro, nrow), :] = jnp.concatenate(
                val_blocks[kind], axis=0)


def _scatter_body(uidx_hbm, vals_hbm, out_hbm,
                  idx_v, val_v, irow_v, vrow_v, fin_v, ob_v, acc_sh):
    c = lax.axis_index("c")
    s = lax.axis_index("s")

    zspan = (2 * 4 * NCPAD) // 16

    @pl.loop(0, SPAN // 16)
    def _(t):
        ob_v[pl.ds(t * 16, 16)] = jnp.zeros((16,), jnp.float32)

    for q in range(zspan // SPAN):
        pltpu.sync_copy(ob_v.at[pl.ds(0, SPAN)],
                        acc_sh.at[pl.ds(s * zspan + q * SPAN, SPAN)])
    plsc.subcore_barrier()

    b_local = s // 8
    kind = (s % 8) // 2
    half = s % 2
    b_glob = c * 2 + b_local
    base_row = half * HALF_ROWS
    kbase = (b_local * 4 + kind) * NCPAD

    for chunk in range(NCHUNK):
        r0 = base_row + chunk * CHUNK
        pltpu.sync_copy(uidx_hbm.at[b_glob, pl.ds(r0, CHUNK), :], idx_v)
        pltpu.sync_copy(vals_hbm.at[b_glob, kind, pl.ds(r0, CHUNK), :], val_v)

        @pl.loop(0, CHUNK)
        def _(rr):
            for q in range(8):
                sl = pl.ds(q * 16, 16)
                irow_v[sl] = idx_v[rr, sl] + kbase
                vrow_v[sl] = val_v[rr, sl]
            pltpu.sync_copy(vrow_v, acc_sh.at[irow_v], add=True)

    plsc.subcore_barrier()

    bf = s // 8
    j = s % 8
    span0 = j * SPAN
    for k4 in range(4):
        pltpu.sync_copy(
            acc_sh.at[pl.ds((bf * 4 + k4) * NCPAD + span0, SPAN)],
            fin_v.at[pl.ds(k4 * SPAN, SPAN)])

    @pl.loop(0, SPAN // 16)
    def _(t):
        wv = fin_v[pl.ds(3 * SPAN + t * 16, 16)]
        m = jnp.maximum(wv, 1.0)
        for ch in range(3):
            ob_v[pl.ds(ch * SPAN + t * 16, 16)] = (
                fin_v[pl.ds(ch * SPAN + t * 16, 16)] / m)

    for ch in range(3):
        pltpu.sync_copy(
            ob_v.at[pl.ds(ch * SPAN, SPAN)],
            out_hbm.at[pl.ds(((c * 2 + bf) * 3 + ch) * NCPAD + span0, SPAN)])


def _make_tc_calls():
    fps = pl.pallas_call(
        _fps_body,
        out_shape=jax.ShapeDtypeStruct((B, 3, K, 1), jnp.float32),
    )
    corners = pl.pallas_call(
        _corner_body,
        grid=(B,),
        in_specs=[
            pl.BlockSpec((1, 3, 64, 128), lambda b: (b, 0, 0, 0)),
            pl.BlockSpec((1, 3, 32, 128), lambda b: (b, 0, 0, 0)),
        ],
        out_specs=[
            pl.BlockSpec((1, TOT_ROWS, 128), lambda b: (b, 0, 0)),
            pl.BlockSpec((1, 4, TOT_ROWS, 128), lambda b: (b, 0, 0, 0)),
        ],
        out_shape=[
            jax.ShapeDtypeStruct((B, TOT_ROWS, 128), jnp.int32),
            jax.ShapeDtypeStruct((B, 4, TOT_ROWS, 128), jnp.float32),
        ],
    )
    return fps, corners


@functools.cache
def _make_scatter_call():
    mesh = plsc.VectorSubcoreMesh(core_axis_name="c", subcore_axis_name="s",
                                  num_cores=2, num_subcores=16)
    scatter = pl.kernel(
        _scatter_body,
        out_type=jax.ShapeDtypeStruct((B * 3 * NCPAD,), jnp.float32),
        mesh=mesh,
        scratch_types=[
            pltpu.VMEM((CHUNK, 128), jnp.int32),
            pltpu.VMEM((CHUNK, 128), jnp.float32),
            pltpu.VMEM((128,), jnp.int32),
            pltpu.VMEM((128,), jnp.float32),
            pltpu.VMEM((4 * SPAN,), jnp.float32),
            pltpu.VMEM((3 * SPAN,), jnp.float32),
            pltpu.VMEM_SHARED((2 * 4 * NCPAD,), jnp.float32),
        ],
    )
    return scatter


_fps_call, _corner_call = _make_tc_calls()


@jax.jit
def kernel(x):
    x4 = x.reshape(B, 3, 64, 128)
    sel = _fps_call(x4)
    sel4 = sel.reshape(B, 3, 32, 128)
    uidx, vals = _corner_call(x4, sel4)
    flat = _make_scatter_call()(uidx, vals).reshape(B, 3, NCPAD)
    outs = []
    for l in range(NLEV):
        g = GS[l]
        outs.append(
            flat[:, :, CELL_OFF[l]:CELL_OFF[l] + g ** 3]
            .reshape(B, 3, g, g, g))
    return tuple(outs)


# speedup vs baseline: 2.0276x; 2.0276x over previous
import functools

import jax
import jax.numpy as jnp
from jax import lax
from jax.experimental import pallas as pl
from jax.experimental.pallas import tpu as pltpu
from jax.experimental.pallas import tpu_sc as plsc

B = 4
N = 8192
K = 4096
NLEV = 5
GS = [32, 16, 8, 4, 2]
NS = [8192, 4096, 2048, 1024, 512]
CELL_OFF = [0, 32768, 36864, 37376, 37440]
NCELL = 37448
NCPAD = 37504
SPAN = NCPAD // 8
UPD_ROWS = [n * 8 // 128 for n in NS]
UPD_ROW_OFF = [0, 512, 768, 896, 960]
TOT_ROWS = 992
HALF_ROWS = TOT_ROWS // 2
CHUNK = 16
NCHUNK = HALF_ROWS // CHUNK


def _fps_body(x_ref, sel_ref):
    p = [[x_ref[b, c].reshape(8, 8, 128) for c in range(3)] for b in range(B)]
    i0 = lax.broadcasted_iota(jnp.int32, (8, 8, 128), 0)
    i1 = lax.broadcasted_iota(jnp.int32, (8, 8, 128), 1)
    i2 = lax.broadcasted_iota(jnp.int32, (8, 8, 128), 2)
    flat = i0 * 1024 + i1 * 128 + i2
    big = jnp.int32(1 << 30)
    ones = jnp.ones((128, 128), jnp.float32)

    def lane_sum_bcast(v8):
        s = jnp.dot(v8, ones, preferred_element_type=jnp.float32,
                    precision=jax.lax.Precision.HIGHEST)
        for sft in (4, 2, 1):
            s = s + pltpu.roll(s, sft, 0)
        return s

    carry0 = []
    for b in range(B):
        l0 = [x_ref[b, c, 0, 0] for c in range(3)]
        for c in range(3):
            sel_ref[b, c, pl.ds(0, 1), :] = jnp.full((1, 1), l0[c], jnp.float32)
        carry0.append((jnp.full((8, 8, 128), 1e10, jnp.float32),
                       *[jnp.full((8, 128), l0[c], jnp.float32)
                         for c in range(3)]))

    def body(i, carry):
        new = []
        for b in range(B):
            dists, lx, ly, lz = carry[b]
            dx = p[b][0] - lx[None]
            dd = dx * dx
            dy = p[b][1] - ly[None]
            dd = dd + dy * dy
            dz = p[b][2] - lz[None]
            dd = dd + dz * dz
            dists = jnp.minimum(dists, dd)
            idx = jnp.argmax(dists).astype(jnp.int32)
            oh2 = flat == idx
            nl = tuple(
                lane_sum_bcast(jnp.sum(jnp.where(oh2, p[b][c], 0.0), axis=0))
                for c in range(3))
            new.append((dists, *nl))
        for b in range(B):
            for c in range(3):
                sel_ref[b, c, pl.ds(i, 1), :] = new[b][1 + c][0:1, 0:1]
        return tuple(new)

    lax.fori_loop(1, K, body, tuple(carry0))


def _corner_body(x_ref, sel_ref, uidx_ref, vals_ref):
    for l in range(NLEV):
        g = GS[l]
        rows = NS[l] // 128
        if l == 0:
            p = [x_ref[0, c] for c in range(3)]
        else:
            p = [sel_ref[0, c, pl.ds(0, rows), :] for c in range(3)]
        gf = float(g)
        r = [(pc + 0.5) * gf - 0.5 for pc in p]
        lo = [jnp.clip(jnp.floor(rc), 0.0, gf - 1.0) for rc in r]
        hi = [jnp.clip(jnp.ceil(rc), 0.0, gf - 1.0) for rc in r]
        idx_blocks = []
        val_blocks = [[], [], [], []]
        for corner in range(8):
            cx = hi[0] if (corner >> 2) & 1 else lo[0]
            cy = hi[1] if (corner >> 1) & 1 else lo[1]
            cz = hi[2] if corner & 1 else lo[2]
            resx = r[0] - cx
            resy = r[1] - cy
            resz = r[2] - cz
            dist = jnp.sqrt(resx * resx + resy * resy + resz * resz)
            w = (dist < 0.87).astype(jnp.float32)
            cell = (cx * (gf * gf) + cy * gf + cz).astype(jnp.int32) + CELL_OFF[l]
            idx_blocks.append(cell)
            val_blocks[0].append(resx * w)
            val_blocks[1].append(resy * w)
            val_blocks[2].append(resz * w)
            val_blocks[3].append(w)
        ro = UPD_ROW_OFF[l]
        nrow = 8 * rows
        uidx_ref[0, pl.ds(ro, nrow), :] = jnp.concatenate(idx_blocks, axis=0)
        for kind in range(4):
            vals_ref[0, kind, pl.ds(ro, nrow), :] = jnp.concatenate(
                val_blocks[kind], axis=0)


def _scatter_body(uidx_hbm, vals_hbm, out_hbm,
                  idx_v, val_v, irow_v, vrow_v, fin_v, ob_v, acc_sh):
    c = lax.axis_index("c")
    s = lax.axis_index("s")

    zspan = (2 * 4 * NCPAD) // 16

    @pl.loop(0, SPAN // 16)
    def _(t):
        ob_v[pl.ds(t * 16, 16)] = jnp.zeros((16,), jnp.float32)

    for q in range(zspan // SPAN):
        pltpu.sync_copy(ob_v.at[pl.ds(0, SPAN)],
                        acc_sh.at[pl.ds(s * zspan + q * SPAN, SPAN)])
    plsc.subcore_barrier()

    b_local = s // 8
    kind = (s % 8) // 2
    half = s % 2
    b_glob = c * 2 + b_local
    base_row = half * HALF_ROWS
    kbase = (b_local * 4 + kind) * NCPAD

    for chunk in range(NCHUNK):
        r0 = base_row + chunk * CHUNK
        pltpu.sync_copy(uidx_hbm.at[b_glob, pl.ds(r0, CHUNK), :], idx_v)
        pltpu.sync_copy(vals_hbm.at[b_glob, kind, pl.ds(r0, CHUNK), :], val_v)

        @pl.loop(0, CHUNK)
        def _(rr):
            for q in range(8):
                sl = pl.ds(q * 16, 16)
                irow_v[sl] = idx_v[rr, sl] + kbase
                vrow_v[sl] = val_v[rr, sl]
            pltpu.sync_copy(vrow_v, acc_sh.at[irow_v], add=True)

    plsc.subcore_barrier()

    bf = s // 8
    j = s % 8
    span0 = j * SPAN
    for k4 in range(4):
        pltpu.sync_copy(
            acc_sh.at[pl.ds((bf * 4 + k4) * NCPAD + span0, SPAN)],
            fin_v.at[pl.ds(k4 * SPAN, SPAN)])

    @pl.loop(0, SPAN // 16)
    def _(t):
        wv = fin_v[pl.ds(3 * SPAN + t * 16, 16)]
        m = jnp.maximum(wv, 1.0)
        for ch in range(3):
            ob_v[pl.ds(ch * SPAN + t * 16, 16)] = (
                fin_v[pl.ds(ch * SPAN + t * 16, 16)] / m)

    for ch in range(3):
        pltpu.sync_copy(
            ob_v.at[pl.ds(ch * SPAN, SPAN)],
            out_hbm.at[pl.ds(((c * 2 + bf) * 3 + ch) * NCPAD + span0, SPAN)])


def _make_tc_calls():
    fps = pl.pallas_call(
        _fps_body,
        out_shape=jax.ShapeDtypeStruct((B, 3, K, 1), jnp.float32),
    )
    corners = pl.pallas_call(
        _corner_body,
        grid=(B,),
        in_specs=[
            pl.BlockSpec((1, 3, 64, 128), lambda b: (b, 0, 0, 0)),
            pl.BlockSpec((1, 3, 32, 128), lambda b: (b, 0, 0, 0)),
        ],
        out_specs=[
            pl.BlockSpec((1, TOT_ROWS, 128), lambda b: (b, 0, 0)),
            pl.BlockSpec((1, 4, TOT_ROWS, 128), lambda b: (b, 0, 0, 0)),
        ],
        out_shape=[
            jax.ShapeDtypeStruct((B, TOT_ROWS, 128), jnp.int32),
            jax.ShapeDtypeStruct((B, 4, TOT_ROWS, 128), jnp.float32),
        ],
    )
    return fps, corners


@functools.cache
def _make_scatter_call():
    mesh = plsc.VectorSubcoreMesh(core_axis_name="c", subcore_axis_name="s",
                                  num_cores=2, num_subcores=16)
    scatter = pl.kernel(
        _scatter_body,
        out_type=jax.ShapeDtypeStruct((B * 3 * NCPAD,), jnp.float32),
        mesh=mesh,
        scratch_types=[
            pltpu.VMEM((CHUNK, 128), jnp.int32),
            pltpu.VMEM((CHUNK, 128), jnp.float32),
            pltpu.VMEM((128,), jnp.int32),
            pltpu.VMEM((128,), jnp.float32),
            pltpu.VMEM((4 * SPAN,), jnp.float32),
            pltpu.VMEM((3 * SPAN,), jnp.float32),
            pltpu.VMEM_SHARED((2 * 4 * NCPAD,), jnp.float32),
        ],
    )
    return scatter


_fps_call, _corner_call = _make_tc_calls()


@jax.jit
def kernel(x):
    x4 = x.reshape(B, 3, 64, 128)
    sel = _fps_call(x4)
    sel4 = sel.reshape(B, 3, 32, 128)
    uidx, vals = _corner_call(x4, sel4)
    flat = _make_scatter_call()(uidx, vals).reshape(B, 3, NCPAD)
    outs = []
    for l in range(NLEV):
        g = GS[l]
        outs.append(
            flat[:, :, CELL_OFF[l]:CELL_OFF[l] + g ** 3]
            .reshape(B, 3, g, g, g))
    return tuple(outs)
